# Initial kernel scaffold; baseline (speedup 1.0000x reference)
import jax
import jax.numpy as jnp
from jax.experimental import pallas as pl


def kernel(atomic_numbers, table):
    raise NotImplementedError("write your pallas kernel here")


# speedup vs baseline: 1.0951x; 1.0951x over previous
import dataclasses
import functools

import jax
import jax.numpy as jnp
from jax import lax
from jax.experimental import pallas as pl
from jax.experimental.pallas import tpu as pltpu
from jax.experimental.pallas import tpu_sc as plsc

B = 1024
N = 200
D = 64
L = 16
NC = 2
NS = 16
NW = NC * NS
B_PER_W = B // NW
TAIL = N - L

_mesh = plsc.VectorSubcoreMesh(core_axis_name="c", subcore_axis_name="s")

_cp = pltpu.CompilerParams(
    needs_layout_passes=False,
    use_tc_tiling_on_sc=False,
)


@functools.partial(
    pl.kernel,
    mesh=_mesh,
    compiler_params=_cp,
    out_type=jax.ShapeDtypeStruct((B, 2 * D, N), jnp.float32),
    scratch_types=[
        pltpu.VMEM((N,), jnp.int32),
        pltpu.VMEM((N, D), jnp.float32),
        pltpu.VMEM((2 * D, N), jnp.float32),
        pltpu.SemaphoreType.DMA,
    ],
)
def _sc_embed(idx_hbm, table_hbm, out_hbm, idx_v, rows_v, tbuf, sem):
    wid = lax.axis_index("s") * NC + lax.axis_index("c")

    zeros16 = jnp.zeros((L,), jnp.float32)
    iota16 = lax.iota(jnp.int32, L)

    @pl.loop(0, D)
    def _(d):
        @pl.loop(0, TAIL, step=L)
        def _(c):
            tbuf[2 * d + 1, pl.ds(c, L)] = zeros16

        tbuf[2 * d + 1, pl.ds(TAIL, L)] = zeros16

    @pl.loop(0, B_PER_W)
    def _(i):
        b = wid * B_PER_W + i
        pltpu.sync_copy(idx_hbm.at[pl.ds(b * N, N)], idx_v)
        pltpu.async_copy(table_hbm.at[idx_v], rows_v, sem).wait()

        def transpose_chunk(c):
            row_ids = iota16 + c
            for d in range(D):
                col_ids = jnp.full((L,), d, jnp.int32)
                tbuf[2 * d, pl.ds(c, L)] = plsc.load_gather(
                    rows_v, [row_ids, col_ids]
                )

        @pl.loop(0, TAIL, step=L)
        def _(c):
            transpose_chunk(c)

        transpose_chunk(TAIL)

        pltpu.sync_copy(tbuf, out_hbm.at[b])


def kernel(atomic_numbers, table):
    idx_flat = atomic_numbers.reshape(-1).astype(jnp.int32)
    out = _sc_embed(idx_flat, table)
    return out.reshape(B, D, 1, 2, 1, 1, N)


# speedup vs baseline: 1.2298x; 1.1230x over previous
import functools

import jax
import jax.numpy as jnp
from jax import lax
from jax.experimental import pallas as pl
from jax.experimental.pallas import tpu as pltpu
from jax.experimental.pallas import tpu_sc as plsc

B = 1024
N = 200
D = 64
L = 16
NC = 2
NS = 16
NW = NC * NS
B_PER_W = B // NW
TAIL = N - L

_mesh = plsc.VectorSubcoreMesh(core_axis_name="c", subcore_axis_name="s")

_cp = pltpu.CompilerParams(
    needs_layout_passes=False,
    use_tc_tiling_on_sc=False,
)


@functools.partial(
    pl.kernel,
    mesh=_mesh,
    compiler_params=_cp,
    out_type=jax.ShapeDtypeStruct((B, 2 * D, N), jnp.float32),
    scratch_types=[
        pltpu.VMEM((B_PER_W, N), jnp.int32),
        pltpu.VMEM((N, D), jnp.float32),
        pltpu.VMEM((N, D), jnp.float32),
        pltpu.VMEM((2 * D, N), jnp.float32),
        pltpu.VMEM((2 * D, N), jnp.float32),
        pltpu.SemaphoreType.DMA,
        pltpu.SemaphoreType.DMA,
        pltpu.SemaphoreType.DMA,
        pltpu.SemaphoreType.DMA,
    ],
)
def _sc_embed(idx_hbm, table_hbm, out_hbm, idx_all, rows0, rows1,
              tbuf0, tbuf1, gsem0, gsem1, osem0, osem1):
    wid = lax.axis_index("s") * NC + lax.axis_index("c")
    base = wid * B_PER_W

    zeros16 = jnp.zeros((L,), jnp.float32)
    iota16 = lax.iota(jnp.int32, L)

    pltpu.sync_copy(idx_hbm.at[pl.ds(base, B_PER_W)], idx_all)

    @pl.loop(0, D)
    def _(d):
        for tbuf in (tbuf0, tbuf1):
            @pl.loop(0, TAIL, step=L)
            def _(c):
                tbuf[2 * d + 1, pl.ds(c, L)] = zeros16

            tbuf[2 * d + 1, pl.ds(TAIL, L)] = zeros16

    def transpose(rows, tbuf):
        def chunk(c):
            row_ids = iota16 + c
            for d in range(D):
                col_ids = jnp.full((L,), d, jnp.int32)
                tbuf[2 * d, pl.ds(c, L)] = plsc.load_gather(
                    rows, [row_ids, col_ids]
                )

        @pl.loop(0, TAIL, step=L)
        def _(c):
            chunk(c)

        chunk(TAIL)

    slots = ((rows0, tbuf0, gsem0, osem0), (rows1, tbuf1, gsem1, osem1))

    pltpu.async_copy(table_hbm.at[idx_all.at[0]], rows0, gsem0)
    pltpu.async_copy(table_hbm.at[idx_all.at[1]], rows1, gsem1)

    @pl.loop(0, B_PER_W, step=2)
    def _(i):
        for s, (rows, tbuf, gsem, osem) in enumerate(slots):
            ii = i + s

            pltpu.make_async_copy(
                table_hbm.at[idx_all.at[0]], rows, gsem).wait()

            @pl.when(ii >= 2)
            def _():
                pltpu.make_async_copy(tbuf, out_hbm.at[base], osem).wait()

            transpose(rows, tbuf)
            pltpu.async_copy(tbuf, out_hbm.at[base + ii], osem)

            @pl.when(ii + 2 < B_PER_W)
            def _():
                pltpu.async_copy(
                    table_hbm.at[idx_all.at[ii + 2]], rows, gsem)

    pltpu.make_async_copy(tbuf0, out_hbm.at[base], osem0).wait()
    pltpu.make_async_copy(tbuf1, out_hbm.at[base], osem1).wait()


def kernel(atomic_numbers, table):
    out = _sc_embed(atomic_numbers.astype(jnp.int32), table)
    return out.reshape(B, D, 1, 2, 1, 1, N)


# speedup vs baseline: 3.9570x; 3.2175x over previous
import functools

import jax
import jax.numpy as jnp
from jax import lax
from jax.experimental import pallas as pl
from jax.experimental.pallas import tpu as pltpu
from jax.experimental.pallas import tpu_sc as plsc

B = 1024
N = 200
D = 64
V = 100000
L = 16
NC = 2
NS = 16
NW = NC * NS
D_PER_W = D // NW
RB = 8
CB = 512
NBLK_C = B // CB
NBLK = (N // RB) * NBLK_C

_mesh = plsc.VectorSubcoreMesh(core_axis_name="c", subcore_axis_name="s")

_cp = pltpu.CompilerParams(
    needs_layout_passes=False,
    use_tc_tiling_on_sc=True,
)


@functools.partial(
    pl.kernel,
    mesh=_mesh,
    compiler_params=_cp,
    out_type=jax.ShapeDtypeStruct((D, 2, N, B), jnp.float32),
    scratch_types=[
        pltpu.VMEM((V,), jnp.float32),
        pltpu.VMEM((RB, CB), jnp.int32),
        pltpu.VMEM((RB, CB), jnp.int32),
        pltpu.VMEM((RB, CB), jnp.float32),
        pltpu.VMEM((RB, CB), jnp.float32),
        pltpu.VMEM((RB, CB), jnp.float32),
        pltpu.SemaphoreType.DMA,
        pltpu.SemaphoreType.DMA,
        pltpu.SemaphoreType.DMA,
        pltpu.SemaphoreType.DMA,
        pltpu.SemaphoreType.DMA,
        pltpu.SemaphoreType.DMA,
    ],
)
def _sc_embed(idx_hbm, table_hbm, out_hbm, trow, iblk0, iblk1,
              oblk0, oblk1, zblk, isem0, isem1, osem0, osem1, zsem, tsem):
    wid = lax.axis_index("s") * NC + lax.axis_index("c")

    zeros16 = jnp.zeros((L,), jnp.float32)

    @pl.loop(0, RB)
    def _(r):
        @pl.loop(0, CB, step=L)
        def _(c):
            zblk[r, pl.ds(c, L)] = zeros16

    def blk_pos(ii):
        r0 = (ii // NBLK_C) * RB
        c0 = (ii % NBLK_C) * CB
        return r0, c0

    def idx_src(ii):
        r0, c0 = blk_pos(ii)
        return idx_hbm.at[pl.ds(r0, RB), pl.ds(c0, CB)]

    slots = ((iblk0, oblk0, isem0, osem0), (iblk1, oblk1, isem1, osem1))

    for dd in range(D_PER_W):
        d = wid * D_PER_W + dd

        pltpu.async_copy(table_hbm.at[d], trow, tsem).wait()

        pltpu.async_copy(idx_src(0), iblk0, isem0)
        pltpu.async_copy(idx_src(1), iblk1, isem1)

        @pl.loop(0, NBLK, step=2)
        def _(i):
            for s, (iblk, oblk, isem, osem) in enumerate(slots):
                ii = i + s
                r0, c0 = blk_pos(ii)
                dst = out_hbm.at[d, 0, pl.ds(r0, RB), pl.ds(c0, CB)]
                zdst = out_hbm.at[d, 1, pl.ds(r0, RB), pl.ds(c0, CB)]

                pltpu.make_async_copy(idx_src(0), iblk, isem).wait()

                @pl.when(ii >= 2)
                def _():
                    pltpu.make_async_copy(oblk, dst, osem).wait()

                @pl.loop(0, CB, step=L)
                def _(c):
                    for r in range(RB):
                        iv = iblk[r, pl.ds(c, L)]
                        oblk[r, pl.ds(c, L)] = plsc.load_gather(trow, [iv])

                pltpu.async_copy(oblk, dst, osem)

                @pl.when(ii >= 1)
                def _():
                    pltpu.make_async_copy(zblk, zdst, zsem).wait()

                pltpu.async_copy(zblk, zdst, zsem)

                @pl.when(ii + 2 < NBLK)
                def _():
                    pltpu.async_copy(idx_src(ii + 2), iblk, isem)

        pltpu.make_async_copy(
            oblk0, out_hbm.at[d, 0, pl.ds(0, RB), pl.ds(0, CB)], osem0).wait()
        pltpu.make_async_copy(
            oblk1, out_hbm.at[d, 0, pl.ds(0, RB), pl.ds(0, CB)], osem1).wait()
        pltpu.make_async_copy(
            zblk, out_hbm.at[d, 1, pl.ds(0, RB), pl.ds(0, CB)], zsem).wait()


def kernel(atomic_numbers, table):
    idx_t = atomic_numbers.T
    table_t = table.T
    out = _sc_embed(idx_t, table_t)
    return jnp.transpose(out, (3, 0, 1, 2)).reshape(B, D, 1, 2, 1, 1, N)


# speedup vs baseline: 6.3810x; 1.6126x over previous
import functools

import jax
import jax.numpy as jnp
from jax import lax
from jax.experimental import pallas as pl
from jax.experimental.pallas import tpu as pltpu
from jax.experimental.pallas import tpu_sc as plsc

B = 1024
N = 200
D = 64
V = 100000
L = 16
NC = 2
NS = 16
NW = NC * NS
D_PER_W = D // NW
RB = 8
CB = 512
NBLK_C = B // CB
NBLK = (N // RB) * NBLK_C

_mesh = plsc.VectorSubcoreMesh(core_axis_name="c", subcore_axis_name="s")

_cp = pltpu.CompilerParams(
    needs_layout_passes=False,
    use_tc_tiling_on_sc=True,
)


@functools.partial(
    pl.kernel,
    mesh=_mesh,
    compiler_params=_cp,
    out_type=jax.ShapeDtypeStruct((D, 2, N, B), jnp.float32),
    scratch_types=[
        pltpu.VMEM((V,), jnp.float32),
        pltpu.VMEM((RB, CB), jnp.int32),
        pltpu.VMEM((RB, CB), jnp.int32),
        pltpu.VMEM((RB, CB), jnp.float32),
        pltpu.VMEM((RB, CB), jnp.float32),
        pltpu.VMEM((RB, CB), jnp.float32),
        pltpu.SemaphoreType.DMA,
        pltpu.SemaphoreType.DMA,
        pltpu.SemaphoreType.DMA,
        pltpu.SemaphoreType.DMA,
        pltpu.SemaphoreType.DMA,
        pltpu.SemaphoreType.DMA,
    ],
)
def _sc_embed(idx_hbm, table_hbm, out_hbm, trow, iblk0, iblk1,
              oblk0, oblk1, zblk, isem0, isem1, osem0, osem1, zsem, tsem):
    wid = lax.axis_index("s") * NC + lax.axis_index("c")

    zeros16 = jnp.zeros((L,), jnp.float32)

    @pl.loop(0, RB)
    def _(r):
        @pl.loop(0, CB, step=L)
        def _(c):
            zblk[r, pl.ds(c, L)] = zeros16

    def blk_pos(ii):
        r0 = (ii // NBLK_C) * RB
        c0 = (ii % NBLK_C) * CB
        return r0, c0

    def idx_src(ii):
        r0, c0 = blk_pos(ii)
        return idx_hbm.at[pl.ds(r0, RB), pl.ds(c0, CB)]

    slots = ((iblk0, oblk0, isem0, osem0), (iblk1, oblk1, isem1, osem1))

    for dd in range(D_PER_W):
        d = wid * D_PER_W + dd

        pltpu.async_copy(table_hbm.at[d], trow, tsem).wait()

        pltpu.async_copy(idx_src(0), iblk0, isem0)
        pltpu.async_copy(idx_src(1), iblk1, isem1)

        @pl.loop(0, NBLK, step=2)
        def _(i):
            for s, (iblk, oblk, isem, osem) in enumerate(slots):
                ii = i + s
                r0, c0 = blk_pos(ii)
                dst = out_hbm.at[d, 0, pl.ds(r0, RB), pl.ds(c0, CB)]
                zdst = out_hbm.at[d, 1, pl.ds(r0, RB), pl.ds(c0, CB)]

                pltpu.make_async_copy(idx_src(0), iblk, isem).wait()

                @pl.when(ii >= 2)
                def _():
                    pltpu.make_async_copy(oblk, dst, osem).wait()

                @plsc.parallel_loop(0, CB, step=L, unroll=2)
                def _(c):
                    for r in range(RB):
                        iv = iblk[r, pl.ds(c, L)]
                        oblk[r, pl.ds(c, L)] = plsc.load_gather(trow, [iv])

                pltpu.async_copy(oblk, dst, osem)

                @pl.when(ii >= 1)
                def _():
                    pltpu.make_async_copy(zblk, zdst, zsem).wait()

                pltpu.async_copy(zblk, zdst, zsem)

                @pl.when(ii + 2 < NBLK)
                def _():
                    pltpu.async_copy(idx_src(ii + 2), iblk, isem)

        pltpu.make_async_copy(
            oblk0, out_hbm.at[d, 0, pl.ds(0, RB), pl.ds(0, CB)], osem0).wait()
        pltpu.make_async_copy(
            oblk1, out_hbm.at[d, 0, pl.ds(0, RB), pl.ds(0, CB)], osem1).wait()
        pltpu.make_async_copy(
            zblk, out_hbm.at[d, 1, pl.ds(0, RB), pl.ds(0, CB)], zsem).wait()


def kernel(atomic_numbers, table):
    idx_t = atomic_numbers.T
    table_t = table.T
    out = _sc_embed(idx_t, table_t)
    return jnp.transpose(out, (3, 0, 1, 2)).reshape(B, D, 1, 2, 1, 1, N)


# speedup vs baseline: 6.4263x; 1.0071x over previous
import functools

import jax
import jax.numpy as jnp
from jax import lax
from jax.experimental import pallas as pl
from jax.experimental.pallas import tpu as pltpu
from jax.experimental.pallas import tpu_sc as plsc

B = 1024
N = 200
D = 64
V = 100000
L = 16
NC = 2
NS = 16
NW = NC * NS
D_PER_W = D // NW
RB = 8
CB = 512
NBLK_C = B // CB
NBLK = (N // RB) * NBLK_C

_mesh = plsc.VectorSubcoreMesh(core_axis_name="c", subcore_axis_name="s")

_cp = pltpu.CompilerParams(
    needs_layout_passes=False,
    use_tc_tiling_on_sc=True,
)


@functools.partial(
    pl.kernel,
    mesh=_mesh,
    compiler_params=_cp,
    out_type=jax.ShapeDtypeStruct((D, 2, N, B), jnp.float32),
    scratch_types=[
        pltpu.VMEM((V,), jnp.float32),
        pltpu.VMEM((RB, CB), jnp.int32),
        pltpu.VMEM((RB, CB), jnp.int32),
        pltpu.VMEM((RB, CB), jnp.float32),
        pltpu.VMEM((RB, CB), jnp.float32),
        pltpu.VMEM((RB, B), jnp.float32),
        pltpu.SemaphoreType.DMA,
        pltpu.SemaphoreType.DMA,
        pltpu.SemaphoreType.DMA,
        pltpu.SemaphoreType.DMA,
        pltpu.SemaphoreType.DMA,
        pltpu.SemaphoreType.DMA,
    ],
)
def _sc_embed(idx_hbm, table_hbm, out_hbm, trow, iblk0, iblk1,
              oblk0, oblk1, zblk, isem0, isem1, osem0, osem1, zsem, tsem):
    wid = lax.axis_index("s") * NC + lax.axis_index("c")

    zeros16 = jnp.zeros((L,), jnp.float32)

    @pl.loop(0, RB)
    def _(r):
        @pl.loop(0, B, step=L)
        def _(c):
            zblk[r, pl.ds(c, L)] = zeros16

    def blk_pos(ii):
        r0 = (ii // NBLK_C) * RB
        c0 = (ii % NBLK_C) * CB
        return r0, c0

    def idx_src(ii):
        r0, c0 = blk_pos(ii)
        return idx_hbm.at[pl.ds(r0, RB), pl.ds(c0, CB)]

    slots = ((iblk0, oblk0, isem0, osem0), (iblk1, oblk1, isem1, osem1))

    pltpu.async_copy(table_hbm.at[wid * D_PER_W], trow, tsem)

    for dd in range(D_PER_W):
        d = wid * D_PER_W + dd

        pltpu.make_async_copy(table_hbm.at[0], trow, tsem).wait()

        pltpu.async_copy(idx_src(0), iblk0, isem0)
        pltpu.async_copy(idx_src(1), iblk1, isem1)

        @pl.loop(0, NBLK, step=2)
        def _(i):
            for s, (iblk, oblk, isem, osem) in enumerate(slots):
                ii = i + s
                r0, c0 = blk_pos(ii)
                dst = out_hbm.at[d, 0, pl.ds(r0, RB), pl.ds(c0, CB)]

                pltpu.make_async_copy(idx_src(0), iblk, isem).wait()

                @pl.when(ii >= 2)
                def _():
                    pltpu.make_async_copy(oblk, dst, osem).wait()

                @plsc.parallel_loop(0, CB, step=L, unroll=4)
                def _(c):
                    for r in range(RB):
                        iv = iblk[r, pl.ds(c, L)]
                        oblk[r, pl.ds(c, L)] = plsc.load_gather(trow, [iv])

                pltpu.async_copy(oblk, dst, osem)

                if s == 0:
                    zdst = out_hbm.at[d, 1, pl.ds(r0, RB), :]

                    @pl.when(ii >= 2)
                    def _():
                        pltpu.make_async_copy(zblk, zdst, zsem).wait()

                    pltpu.async_copy(zblk, zdst, zsem)

                @pl.when(ii + 2 < NBLK)
                def _():
                    pltpu.async_copy(idx_src(ii + 2), iblk, isem)

        if dd + 1 < D_PER_W:
            pltpu.async_copy(table_hbm.at[d + 1], trow, tsem)

        pltpu.make_async_copy(
            oblk0, out_hbm.at[d, 0, pl.ds(0, RB), pl.ds(0, CB)], osem0).wait()
        pltpu.make_async_copy(
            oblk1, out_hbm.at[d, 0, pl.ds(0, RB), pl.ds(0, CB)], osem1).wait()
        pltpu.make_async_copy(
            zblk, out_hbm.at[d, 1, pl.ds(0, RB), :], zsem).wait()


def kernel(atomic_numbers, table):
    idx_t = atomic_numbers.T
    table_t = table.T
    out = _sc_embed(idx_t, table_t)
    return jnp.transpose(out, (3, 0, 1, 2)).reshape(B, D, 1, 2, 1, 1, N)


# speedup vs baseline: 10.5349x; 1.6393x over previous
import functools

import jax
import jax.numpy as jnp
from jax import lax
from jax.experimental import pallas as pl
from jax.experimental.pallas import tpu as pltpu
from jax.experimental.pallas import tpu_sc as plsc

B = 1024
N = 200
D = 64
V = 100000
L = 16
NC = 2
NS = 16
NW = NC * NS
D_PER_W = D // NW
RB = 8
CB = 256
NBLK_C = B // CB
NBLK = (N // RB) * NBLK_C

_mesh = plsc.VectorSubcoreMesh(core_axis_name="c", subcore_axis_name="s")

_cp = pltpu.CompilerParams(
    needs_layout_passes=False,
    use_tc_tiling_on_sc=True,
)


@functools.partial(
    pl.kernel,
    mesh=_mesh,
    compiler_params=_cp,
    out_type=jax.ShapeDtypeStruct((D, 2, N, B), jnp.float32),
    scratch_types=[
        pltpu.VMEM((V,), jnp.float32),
        pltpu.VMEM((RB, CB), jnp.int32),
        pltpu.VMEM((RB, CB), jnp.int32),
        pltpu.VMEM((RB, CB), jnp.float32),
        pltpu.VMEM((RB, CB), jnp.float32),
        pltpu.VMEM((RB, B), jnp.float32),
        pltpu.SemaphoreType.DMA,
        pltpu.SemaphoreType.DMA,
        pltpu.SemaphoreType.DMA,
        pltpu.SemaphoreType.DMA,
        pltpu.SemaphoreType.DMA,
        pltpu.SemaphoreType.DMA,
        pltpu.VMEM_SHARED((N, B), jnp.int32),
        pltpu.SemaphoreType.DMA,
    ],
)
def _sc_embed(idx_hbm, table_hbm, out_hbm, trow, iblk0, iblk1,
              oblk0, oblk1, zblk, isem0, isem1, osem0, osem1, zsem, tsem,
              idx_sh, lsem):
    sid = lax.axis_index("s")
    wid = sid * NC + lax.axis_index("c")

    zeros16 = jnp.zeros((L,), jnp.float32)

    @pl.loop(0, RB)
    def _(r):
        @pl.loop(0, B, step=L)
        def _(c):
            zblk[r, pl.ds(c, L)] = zeros16

    def blk_pos(ii):
        r0 = (ii // NBLK_C) * RB
        c0 = (ii % NBLK_C) * CB
        return r0, c0

    def idx_src(ii):
        r0, c0 = blk_pos(ii)
        return idx_sh.at[pl.ds(r0, RB), pl.ds(c0, CB)]

    slots = ((iblk0, oblk0, isem0, osem0), (iblk1, oblk1, isem1, osem1))

    pltpu.async_copy(table_hbm.at[wid * D_PER_W], trow, tsem)

    @pl.when(sid == 0)
    def _():
        pltpu.async_copy(idx_hbm, idx_sh, lsem).wait()

    plsc.subcore_barrier()

    for dd in range(D_PER_W):
        d = wid * D_PER_W + dd

        pltpu.make_async_copy(table_hbm.at[0], trow, tsem).wait()

        pltpu.async_copy(idx_src(0), iblk0, isem0)
        pltpu.async_copy(idx_src(1), iblk1, isem1)

        @pl.loop(0, NBLK, step=2)
        def _(i):
            for s, (iblk, oblk, isem, osem) in enumerate(slots):
                ii = i + s
                r0, c0 = blk_pos(ii)
                dst = out_hbm.at[d, 0, pl.ds(r0, RB), pl.ds(c0, CB)]

                pltpu.make_async_copy(idx_src(0), iblk, isem).wait()

                @pl.when(ii >= 2)
                def _():
                    pltpu.make_async_copy(oblk, dst, osem).wait()

                @plsc.parallel_loop(0, CB, step=L, unroll=4)
                def _(c):
                    for r in range(RB):
                        iv = iblk[r, pl.ds(c, L)]
                        oblk[r, pl.ds(c, L)] = plsc.load_gather(trow, [iv])

                pltpu.async_copy(oblk, dst, osem)

                if s == 0:
                    zdst = out_hbm.at[d, 1, pl.ds(r0, RB), :]
                    first_col = (ii % NBLK_C) == 0

                    @pl.when(first_col & (ii >= NBLK_C))
                    def _():
                        pltpu.make_async_copy(zblk, zdst, zsem).wait()

                    @pl.when(first_col)
                    def _():
                        pltpu.async_copy(zblk, zdst, zsem)

                @pl.when(ii + 2 < NBLK)
                def _():
                    pltpu.async_copy(idx_src(ii + 2), iblk, isem)

        if dd + 1 < D_PER_W:
            pltpu.async_copy(table_hbm.at[d + 1], trow, tsem)

        pltpu.make_async_copy(
            oblk0, out_hbm.at[d, 0, pl.ds(0, RB), pl.ds(0, CB)], osem0).wait()
        pltpu.make_async_copy(
            oblk1, out_hbm.at[d, 0, pl.ds(0, RB), pl.ds(0, CB)], osem1).wait()
        pltpu.make_async_copy(
            zblk, out_hbm.at[d, 1, pl.ds(0, RB), :], zsem).wait()


def kernel(atomic_numbers, table):
    idx_t = atomic_numbers.T
    table_t = table.T
    out = _sc_embed(idx_t, table_t)
    return jnp.transpose(out, (3, 0, 1, 2)).reshape(B, D, 1, 2, 1, 1, N)
